# baseline (speedup 1.0000x reference)
import functools

import jax
import jax.numpy as jnp
from jax import lax
from jax.experimental import pallas as pl
from jax.experimental.pallas import tpu as pltpu
from jax.experimental.pallas import tpu_sc as plsc

NC = 2
NS = 16
L = 16
NW = NC * NS

B = 16384
D = 32
BPW = B // NW
IDXC = 128
NCH = BPW // IDXC
NG = BPW // L


def _gmf_body(uid_hbm, iid_hbm, ut_hbm, it_hbm, w_hbm, b_hbm, out_hbm,
              uid_v, iid_v, urows_v, vrows_v, w_v, b_v, part_v, res_v,
              sem_u, sem_v):
  wid = lax.axis_index("s") * NC + lax.axis_index("c")
  base = wid * BPW

  pltpu.sync_copy(uid_hbm.at[pl.ds(wid * NCH, NCH)], uid_v)
  pltpu.sync_copy(iid_hbm.at[pl.ds(wid * NCH, NCH)], iid_v)

  cus, cvs = [], []
  for j in range(NCH):
    cus.append(pltpu.async_copy(
        ut_hbm.at[uid_v.at[j]], urows_v.at[pl.ds(j * IDXC, IDXC)], sem_u))
    cvs.append(pltpu.async_copy(
        it_hbm.at[iid_v.at[j]], vrows_v.at[pl.ds(j * IDXC, IDXC)], sem_v))
  pltpu.sync_copy(w_hbm, w_v)
  pltpu.sync_copy(b_hbm, b_v)

  w0 = w_v[pl.ds(0, L)]
  w1 = w_v[pl.ds(L, L)]
  iota = lax.iota(jnp.int32, L)
  scat = iota * BPW

  def row(r, carry):
    u0 = urows_v[r, pl.ds(0, L)]
    u1 = urows_v[r, pl.ds(L, L)]
    v0 = vrows_v[r, pl.ds(0, L)]
    v1 = vrows_v[r, pl.ds(L, L)]
    t = u0 * v0 * w0 + u1 * v1 * w1
    plsc.store_scatter(part_v, [scat + r], t)
    return carry

  for j in range(NCH):
    cus[j].wait()
    cvs[j].wait()
    lax.fori_loop(j * IDXC, (j + 1) * IDXC, row, 0)

  bvec = b_v[...]

  def group(g, carry):
    acc = bvec
    for l in range(L):
      acc = acc + part_v[pl.ds(l * BPW + g * L, L)]
    res_v[pl.ds(g * L, L)] = 1.0 / (1.0 + jnp.exp(-acc))
    return carry

  lax.fori_loop(0, NG, group, 0)
  pltpu.sync_copy(res_v, out_hbm.at[pl.ds(base, BPW)])


@jax.jit
def _gmf_call(uid2d, iid2d, user_table, item_table, w_flat, b16):
  run = pl.kernel(
      _gmf_body,
      out_type=jax.ShapeDtypeStruct((B,), jnp.float32),
      mesh=plsc.VectorSubcoreMesh(
          core_axis_name="c", subcore_axis_name="s",
          num_cores=NC, num_subcores=NS),
      scratch_types=[
          pltpu.VMEM((NCH, IDXC), jnp.int32),
          pltpu.VMEM((NCH, IDXC), jnp.int32),
          pltpu.VMEM((BPW, D), jnp.float32),
          pltpu.VMEM((BPW, D), jnp.float32),
          pltpu.VMEM((D,), jnp.float32),
          pltpu.VMEM((L,), jnp.float32),
          pltpu.VMEM((L * BPW,), jnp.float32),
          pltpu.VMEM((BPW,), jnp.float32),
          pltpu.SemaphoreType.DMA,
          pltpu.SemaphoreType.DMA,
      ],
      compiler_params=pltpu.CompilerParams(
          needs_layout_passes=False, use_tc_tiling_on_sc=False),
  )
  return run(uid2d, iid2d, user_table, item_table, w_flat, b16)


def kernel(user_id, item_id, user_table, item_table, W, b):
  uid2d = user_id.astype(jnp.int32).reshape(B // IDXC, IDXC)
  iid2d = item_id.astype(jnp.int32).reshape(B // IDXC, IDXC)
  w_flat = W.reshape(D).astype(jnp.float32)
  b16 = jnp.broadcast_to(b.astype(jnp.float32), (L,))
  out = _gmf_call(uid2d, iid2d, user_table, item_table, w_flat, b16)
  return out.reshape(B, 1)


# speedup vs baseline: 3.8349x; 3.8349x over previous
import functools

import jax
import jax.numpy as jnp
from jax import lax
from jax.experimental import pallas as pl
from jax.experimental.pallas import tpu as pltpu
from jax.experimental.pallas import tpu_sc as plsc

NC = 2
NS = 16
L = 16
NW = NC * NS

B = 16384
D = 32
BPW = B // NW
NG = BPW // L
NBUF = 4
TW = 128


def _gmf_body(uid_hbm, iid_hbm, ut_hbm, it_hbm, w_hbm, b_hbm, out_hbm,
              uid_sm, iid_sm, idst_v, idst2_v, ubuf_v, vbuf_v, w_v, b_v,
              part_v, res_v,
              sem_u, sem_v):
  wid = lax.axis_index("s") * NC + lax.axis_index("c")
  base = wid * BPW

  pltpu.sync_copy(uid_hbm.at[pl.ds(base, BPW)], idst_v)
  pltpu.sync_copy(iid_hbm.at[pl.ds(base, BPW)], idst2_v)

  def spill(g, carry):
    uvec = idst_v[pl.ds(g * L, L)]
    ivec = idst2_v[pl.ds(g * L, L)]
    for k in range(L):
      uid_sm[g * L + k] = uvec[k]
      iid_sm[g * L + k] = ivec[k]
    return carry

  lax.fori_loop(0, NG, spill, 0)
  pltpu.sync_copy(w_hbm, w_v)
  pltpu.sync_copy(b_hbm, b_v)

  w0 = w_v[pl.ds(0, L)]
  w1 = w_v[pl.ds(L, L)]
  iota = lax.iota(jnp.int32, L)
  d_lo = iota
  d_hi = iota + L
  scat = iota * BPW

  def fire(r, slot):
    ublk = pl.multiple_of((uid_sm[r] >> 7) * TW, TW)
    iblk = pl.multiple_of((iid_sm[r] >> 7) * TW, TW)
    pltpu.async_copy(ut_hbm.at[:, pl.ds(ublk, TW)], ubuf_v.at[slot],
                     sem_u.at[slot])
    pltpu.async_copy(it_hbm.at[:, pl.ds(iblk, TW)], vbuf_v.at[slot],
                     sem_v.at[slot])

  for s in range(NBUF):
    fire(s, s)

  def row(r, carry):
    slot = lax.rem(r, NBUF)
    pltpu.make_async_copy(ut_hbm.at[:, pl.ds(0, TW)], ubuf_v.at[slot],
                          sem_u.at[slot]).wait()
    pltpu.make_async_copy(it_hbm.at[:, pl.ds(0, TW)], vbuf_v.at[slot],
                          sem_v.at[slot]).wait()
    uc = jnp.full((L,), uid_sm[r] & (TW - 1), jnp.int32)
    ic = jnp.full((L,), iid_sm[r] & (TW - 1), jnp.int32)
    u0 = plsc.load_gather(ubuf_v.at[slot], [d_lo, uc])
    u1 = plsc.load_gather(ubuf_v.at[slot], [d_hi, uc])
    v0 = plsc.load_gather(vbuf_v.at[slot], [d_lo, ic])
    v1 = plsc.load_gather(vbuf_v.at[slot], [d_hi, ic])
    t = u0 * v0 * w0 + u1 * v1 * w1
    plsc.store_scatter(part_v, [scat + r], t)

    @pl.when(r < BPW - NBUF)
    def _():
      fire(r + NBUF, slot)

    return carry

  lax.fori_loop(0, BPW, row, 0)

  bvec = b_v[...]

  def group(g, carry):
    acc = bvec
    for l in range(L):
      acc = acc + part_v[pl.ds(l * BPW + g * L, L)]
    res_v[pl.ds(g * L, L)] = 1.0 / (1.0 + jnp.exp(-acc))
    return carry

  lax.fori_loop(0, NG, group, 0)
  pltpu.sync_copy(res_v, out_hbm.at[pl.ds(base, BPW)])


@jax.jit
def _gmf_call(user_id, item_id, ut_t, it_t, w_flat, b16):
  run = pl.kernel(
      _gmf_body,
      out_type=jax.ShapeDtypeStruct((B,), jnp.float32),
      mesh=plsc.VectorSubcoreMesh(
          core_axis_name="c", subcore_axis_name="s",
          num_cores=NC, num_subcores=NS),
      scratch_types=[
          pltpu.SMEM((BPW,), jnp.int32),
          pltpu.SMEM((BPW,), jnp.int32),
          pltpu.VMEM((BPW,), jnp.int32),
          pltpu.VMEM((BPW,), jnp.int32),
          pltpu.VMEM((NBUF, D, TW), jnp.float32),
          pltpu.VMEM((NBUF, D, TW), jnp.float32),
          pltpu.VMEM((D,), jnp.float32),
          pltpu.VMEM((L,), jnp.float32),
          pltpu.VMEM((L * BPW,), jnp.float32),
          pltpu.VMEM((BPW,), jnp.float32),
          pltpu.SemaphoreType.DMA((NBUF,)),
          pltpu.SemaphoreType.DMA((NBUF,)),
      ],
      compiler_params=pltpu.CompilerParams(needs_layout_passes=False),
  )
  return run(user_id, item_id, ut_t, it_t, w_flat, b16)


def kernel(user_id, item_id, user_table, item_table, W, b):
  ut_t = user_table.T
  it_t = item_table.T
  w_flat = W.reshape(D).astype(jnp.float32)
  b16 = jnp.broadcast_to(b.astype(jnp.float32), (L,))
  out = _gmf_call(user_id.astype(jnp.int32), item_id.astype(jnp.int32),
                  ut_t, it_t, w_flat, b16)
  return out.reshape(B, 1)


# speedup vs baseline: 5.3075x; 1.3840x over previous
import functools

import jax
import jax.numpy as jnp
from jax import lax
from jax.experimental import pallas as pl
from jax.experimental.pallas import tpu as pltpu
from jax.experimental.pallas import tpu_sc as plsc

NC = 2
NS = 16
L = 16
NW = NC * NS

B = 16384
D = 32
BPW = B // NW
NG = BPW // L
TW = 128
NBLK_TOT = 7813
NBLK_LO = NBLK_TOT // NW
NBLK_EXTRA = NBLK_TOT - NBLK_LO * NW
NBUF = 8
NOUT = 16
HMAX = 1024
HSM = 896


def _gather_body(uid_hbm, iid_hbm, ut_hbm, it_hbm, urows_out, vrows_out,
                 ids_v, hits_v, blk_v, stage_v, starts_sm, next_sm, hits2_sm,
                 sem_blk, sem_out):
  wid = lax.axis_index("s") * NC + lax.axis_index("c")
  lo = wid * NBLK_LO + jnp.minimum(wid, NBLK_EXTRA)
  nblk = NBLK_LO + (wid < NBLK_EXTRA).astype(jnp.int32)

  iota = lax.iota(jnp.int32, L)
  d_lo = iota
  d_hi = iota + L

  def one_table(ids_hbm, tab_hbm, rows_out):
    pltpu.sync_copy(ids_hbm, ids_v)

    def scan(g, off):
      ids = ids_v[pl.ds(g * L, L)]
      blk = lax.shift_right_logical(ids, 7)
      m = (blk >= lo) & (blk < lo + nblk)
      packed = ((blk - lo) << 21) | ((ids & (TW - 1)) << 14) | (g * L + iota)
      plsc.store_compressed(hits_v.at[pl.ds(off, L)], packed, mask=m)
      cnt = plsc.all_reduce_population_count(m)
      return off + cnt[0]

    htot = lax.fori_loop(0, B // L, scan, 0)
    nv = (htot + L - 1) // L

    def zero(i, c):
      next_sm[i] = 0
      return c
    lax.fori_loop(0, 256, zero, 0)

    def count(g, c):
      hv = hits_v[pl.ds(g * L, L)]
      for k in range(L):
        @pl.when(g * L + k < htot)
        def _():
          blkr = hv[k] >> 21
          next_sm[blkr] = next_sm[blkr] + 1
      return c
    lax.fori_loop(0, nv, count, 0)

    def prefix(i, acc):
      s = next_sm[i]
      starts_sm[i] = acc
      next_sm[i] = acc
      return acc + s
    lax.fori_loop(0, 256, prefix, 0)

    def place(g, c):
      hv = hits_v[pl.ds(g * L, L)]
      for k in range(L):
        @pl.when(g * L + k < htot)
        def _():
          h = hv[k]
          blkr = h >> 21
          pos = next_sm[blkr]
          next_sm[blkr] = pos + 1
          @pl.when(pos < HSM)
          def _():
            hits2_sm[pos] = h
      return c
    lax.fori_loop(0, nv, place, 0)

    def fire(j, slot):
      off = pl.multiple_of((lo + j) * TW, TW)
      pltpu.async_copy(tab_hbm.at[:, pl.ds(off, TW)], blk_v.at[slot],
                       sem_blk.at[slot])

    for s in range(NBUF):
      fire(s, s)

    def block(j, c):
      slot = lax.rem(j, NBUF)
      pltpu.make_async_copy(tab_hbm.at[:, pl.ds(0, TW)], blk_v.at[slot],
                            sem_blk.at[slot]).wait()
      t0 = jnp.minimum(starts_sm[j], HSM)
      t1 = jnp.minimum(starts_sm[j + 1], HSM)

      def hit(t, c2):
        h = hits2_sm[t]
        cc = jnp.full((L,), (h >> 14) & (TW - 1), jnp.int32)
        bidx = h & (B - 1)
        ss = lax.rem(t, NOUT)

        @pl.when(t >= NOUT)
        def _():
          pltpu.make_async_copy(stage_v.at[ss], rows_out.at[pl.ds(0, D)],
                                sem_out.at[ss]).wait()

        u0 = plsc.load_gather(blk_v.at[slot], [d_lo, cc])
        u1 = plsc.load_gather(blk_v.at[slot], [d_hi, cc])
        stage_v[ss, pl.ds(0, L)] = u0
        stage_v[ss, pl.ds(L, L)] = u1
        pltpu.async_copy(stage_v.at[ss], rows_out.at[pl.ds(bidx * D, D)],
                         sem_out.at[ss])
        return c2

      lax.fori_loop(t0, t1, hit, 0)

      @pl.when(j + NBUF < nblk)
      def _():
        fire2 = pl.multiple_of((lo + j + NBUF) * TW, TW)
        pltpu.async_copy(tab_hbm.at[:, pl.ds(fire2, TW)], blk_v.at[slot],
                         sem_blk.at[slot])
      return c

    lax.fori_loop(0, nblk, block, 0)

    def drain(s, c):
      t = htot - 1 - s
      @pl.when(t >= 0)
      def _():
        pltpu.make_async_copy(stage_v.at[lax.rem(t, NOUT)],
                              rows_out.at[pl.ds(0, D)],
                              sem_out.at[lax.rem(t, NOUT)]).wait()
      return c
    lax.fori_loop(0, NOUT, drain, 0)

  one_table(uid_hbm, ut_hbm, urows_out)
  one_table(iid_hbm, it_hbm, vrows_out)


def _dot_body(urows_f, vrows_f, w_hbm, b_hbm, out_hbm,
              u_v, v_v, w_v, b_v, part_v, res_v):
  wid = lax.axis_index("s") * NC + lax.axis_index("c")
  base = wid * BPW

  pltpu.sync_copy(urows_f.at[pl.ds(base * D, BPW * D)], u_v)
  pltpu.sync_copy(vrows_f.at[pl.ds(base * D, BPW * D)], v_v)
  pltpu.sync_copy(w_hbm, w_v)
  pltpu.sync_copy(b_hbm, b_v)

  w0 = w_v[pl.ds(0, L)]
  w1 = w_v[pl.ds(L, L)]
  iota = lax.iota(jnp.int32, L)
  scat = iota * BPW

  def row(r, carry):
    u0 = u_v[pl.ds(r * D, L)]
    u1 = u_v[pl.ds(r * D + L, L)]
    v0 = v_v[pl.ds(r * D, L)]
    v1 = v_v[pl.ds(r * D + L, L)]
    t = u0 * v0 * w0 + u1 * v1 * w1
    plsc.store_scatter(part_v, [scat + r], t)
    return carry

  lax.fori_loop(0, BPW, row, 0)

  bvec = b_v[...]

  def group(g, carry):
    acc = bvec
    for l in range(L):
      acc = acc + part_v[pl.ds(l * BPW + g * L, L)]
    res_v[pl.ds(g * L, L)] = 1.0 / (1.0 + jnp.exp(-acc))
    return carry

  lax.fori_loop(0, NG, group, 0)
  pltpu.sync_copy(res_v, out_hbm.at[pl.ds(base, BPW)])


def _mesh():
  return plsc.VectorSubcoreMesh(
      core_axis_name="c", subcore_axis_name="s",
      num_cores=NC, num_subcores=NS)


@jax.jit
def _gmf_call(user_id, item_id, ut_t, it_t, w_flat, b16):
  gather = pl.kernel(
      _gather_body,
      out_type=(jax.ShapeDtypeStruct((B * D,), jnp.float32),
                jax.ShapeDtypeStruct((B * D,), jnp.float32)),
      mesh=_mesh(),
      scratch_types=[
          pltpu.VMEM((B,), jnp.int32),
          pltpu.VMEM((HMAX,), jnp.int32),
          pltpu.VMEM((NBUF, D, TW), jnp.float32),
          pltpu.VMEM((NOUT, D), jnp.float32),
          pltpu.SMEM((256,), jnp.int32),
          pltpu.SMEM((256,), jnp.int32),
          pltpu.SMEM((HSM,), jnp.int32),
          pltpu.SemaphoreType.DMA((NBUF,)),
          pltpu.SemaphoreType.DMA((NOUT,)),
      ],
      compiler_params=pltpu.CompilerParams(needs_layout_passes=False),
  )
  urows, vrows = gather(user_id, item_id, ut_t, it_t)

  dot = pl.kernel(
      _dot_body,
      out_type=jax.ShapeDtypeStruct((B,), jnp.float32),
      mesh=_mesh(),
      scratch_types=[
          pltpu.VMEM((BPW * D,), jnp.float32),
          pltpu.VMEM((BPW * D,), jnp.float32),
          pltpu.VMEM((D,), jnp.float32),
          pltpu.VMEM((L,), jnp.float32),
          pltpu.VMEM((L * BPW,), jnp.float32),
          pltpu.VMEM((BPW,), jnp.float32),
      ],
      compiler_params=pltpu.CompilerParams(needs_layout_passes=False),
  )
  return dot(urows, vrows, w_flat, b16)


def kernel(user_id, item_id, user_table, item_table, W, b):
  ut_t = user_table.T
  it_t = item_table.T
  w_flat = W.reshape(D).astype(jnp.float32)
  b16 = jnp.broadcast_to(b.astype(jnp.float32), (L,))
  out = _gmf_call(user_id.astype(jnp.int32), item_id.astype(jnp.int32),
                  ut_t, it_t, w_flat, b16)
  return out.reshape(B, 1)
